# baseline (speedup 1.0000x reference)
import jax
import jax.numpy as jnp
from jax.experimental import pallas as pl
from jax.experimental.pallas import tpu as pltpu

_B = 512
_K = 10000
_D = 32
_KB = 1024
_NK = (_K + _KB - 1) // _KB
_TAIL = _K - (_NK - 1) * _KB


def _mm6t_kernel(u, bus, uu, ub, uc, ucat,
                 wu, wb, wc, wcat,
                 bu, bb, bc, bcat,
                 out):
    k = pl.program_id(0)
    xs = (u, bus, uu, ub, uc, ucat)
    ws = (wu, wb, wu, wb, wc, wcat)

    @pl.when(k == 0)
    def _init():
        for i, b in enumerate((bu, bb, bu, bb, bc, bcat)):
            out[_D * i:_D * (i + 1), :] = jnp.broadcast_to(b[...], (_D, _B))

    def accum(n):
        for i in range(6):
            out[_D * i:_D * (i + 1), :] += jnp.dot(
                ws[i][:, 0:n], xs[i][0:n, :],
                preferred_element_type=jnp.float32)

    @pl.when(k < _NK - 1)
    def _full():
        accum(_KB)

    @pl.when(k == _NK - 1)
    def _tail():
        accum(_TAIL)


def kernel(users, businesses, user_user_neigh, user_business_neigh,
           user_city_neigh, user_category_neigh,
           business_business_neigh, business_user_neigh,
           business_city_neigh, business_category_neigh,
           W_user, b_user, W_business, b_business,
           W_city, b_city, W_category, b_category):
    x_spec = pl.BlockSpec((_KB, _B), lambda k: (k, 0))
    w_spec = pl.BlockSpec((_D, _KB), lambda k: (0, k))
    b_spec = pl.BlockSpec((_D, 1), lambda k: (0, 0))

    out = pl.pallas_call(
        _mm6t_kernel,
        grid=(_NK,),
        in_specs=[x_spec] * 6 + [w_spec] * 4 + [b_spec] * 4,
        out_specs=pl.BlockSpec((6 * _D, _B), lambda k: (0, 0)),
        out_shape=jax.ShapeDtypeStruct((6 * _D, _B), jnp.float32),
        compiler_params=pltpu.CompilerParams(
            dimension_semantics=("arbitrary",)),
    )(users.T, businesses.T, user_user_neigh.T, user_business_neigh.T,
      user_city_neigh.T, user_category_neigh.T,
      W_user.T, W_business.T, W_city.T, W_category.T,
      b_user.reshape(_D, 1), b_business.reshape(_D, 1),
      b_city.reshape(_D, 1), b_category.reshape(_D, 1))

    return out.reshape(6, _D, _B).transpose(0, 2, 1)
